# baseline (speedup 1.0000x reference)
import jax
import jax.numpy as jnp
from jax import lax
from jax.experimental import pallas as pl
from jax.experimental.pallas import tpu as pltpu
from jax.experimental.pallas import tpu_sc as plsc

NC = 2
NS = 16
L = 16
ROWS = 16
COLS = 1048576
NBINS = 100
PAD = 128
CH = 16384
NCH = COLS // CH


MMW = 65536
MMG = COLS // MMW


def _mm_body(o_ref, t_ref, lo_ref, scale_ref, amin_o, amax_o, amin_t, amax_t):
    i = pl.program_id(0)

    @pl.when(i == 0)
    def _():
        amin_o[...] = jnp.full((ROWS, 128), jnp.inf, jnp.float32)
        amax_o[...] = jnp.full((ROWS, 128), -jnp.inf, jnp.float32)
        amin_t[...] = jnp.full((ROWS, 128), jnp.inf, jnp.float32)
        amax_t[...] = jnp.full((ROWS, 128), -jnp.inf, jnp.float32)

    xo = o_ref[...]
    xt = t_ref[...]
    bc = lambda v: jnp.broadcast_to(v, (ROWS, 128))
    amin_o[...] = jnp.minimum(amin_o[...], bc(xo.min(axis=1, keepdims=True)))
    amax_o[...] = jnp.maximum(amax_o[...], bc(xo.max(axis=1, keepdims=True)))
    amin_t[...] = jnp.minimum(amin_t[...], bc(xt.min(axis=1, keepdims=True)))
    amax_t[...] = jnp.maximum(amax_t[...], bc(xt.max(axis=1, keepdims=True)))

    @pl.when(i == MMG - 1)
    def _():
        lo_o = amin_o[...]
        hi_o = amax_o[...]
        glo = amin_t[...].min()
        ghi = amax_t[...].max()
        ones = jnp.ones((ROWS, 128), jnp.float32)
        lo_ref[0:ROWS, :] = lo_o
        lo_ref[ROWS:2 * ROWS, :] = glo * ones
        scale_ref[0:ROWS, :] = jnp.float32(NBINS) / (hi_o - lo_o)
        scale_ref[ROWS:2 * ROWS, :] = (jnp.float32(NBINS) / (ghi - glo)) * ones


_mm_tc = pl.pallas_call(
    _mm_body,
    grid=(MMG,),
    in_specs=[
        pl.BlockSpec((ROWS, MMW), lambda i: (0, i)),
        pl.BlockSpec((ROWS, MMW), lambda i: (0, i)),
    ],
    out_specs=[
        pl.BlockSpec((2 * ROWS, 128), lambda i: (0, 0)),
        pl.BlockSpec((2 * ROWS, 128), lambda i: (0, 0)),
    ],
    out_shape=[
        jax.ShapeDtypeStruct((2 * ROWS, 128), jnp.float32),
        jax.ShapeDtypeStruct((2 * ROWS, 128), jnp.float32),
    ],
    scratch_shapes=[pltpu.VMEM((ROWS, 128), jnp.float32)] * 4,
)


def _sc_body(out_hbm, tgt_hbm, lo_hbm, scale_hbm, ph_hbm, tp_hbm,
             buf0, buf1, hist, red, prm, spmem,
             semA0, semA1, semB0, semB1):
    cid = lax.axis_index("c")
    sid = lax.axis_index("s")
    bufs = (buf0, buf1)
    semA = (semA0, semA1)
    semB = (semB0, semB1)
    prow = cid * NS + sid

    pltpu.sync_copy(lo_hbm.at[prow, pl.ds(0, L)], prm)
    lo = prm[...]
    pltpu.sync_copy(scale_hbm.at[prow, pl.ds(0, L)], prm)
    scale = prm[...]

    zvec = jnp.zeros((L,), jnp.float32)

    @plsc.parallel_loop(0, PAD, unroll=8)
    def _(k):
        hist[pl.ds(pl.multiple_of(k * L, 8), L)] = zvec

    lane = lax.broadcasted_iota(jnp.int32, (L,), 0)
    ones = jnp.ones((L,), jnp.float32)

    def process(buf):
        @plsc.parallel_loop(0, CH // L, unroll=4)
        def _(i):
            x = buf[pl.ds(pl.multiple_of(i * L, 8), L)]
            t = (x - lo) * scale
            b = t.astype(jnp.int32)
            addr = (b << 4) | lane
            plsc.addupdate_scatter(hist, [addr], ones)

    def _pipe(src_hbm):
        def startA(j, p):
            off = pl.multiple_of(j * CH, 8)
            return pltpu.async_copy(src_hbm.at[sid, pl.ds(off, CH)],
                                    spmem.at[p, sid], semA[p])

        def startB(p):
            return pltpu.async_copy(spmem.at[p, sid], bufs[p], semB[p])

        dA = [startA(0, 0), startA(1, 1)]
        dA[0].wait()
        dB = [startB(0), None]
        for j in range(NCH):
            p = j % 2
            if j + 1 < NCH:
                dA[1 - p].wait()
                dB[1 - p] = startB(1 - p)
            dB[p].wait()
            if j + 2 < NCH:
                dA[p] = startA(j + 2, p)
            process(bufs[p])

    @pl.when(cid == 0)
    def _():
        _pipe(out_hbm)

    @pl.when(cid == 1)
    def _():
        _pipe(tgt_hbm)

    kidx = lax.broadcasted_iota(jnp.int32, (L,), 0) * L
    for g in range(PAD // L):
        acc = jnp.zeros((L,), jnp.float32)
        for l in range(L):
            acc = acc + plsc.load_gather(hist, [kidx + (g * L * L + l)])
        red[pl.ds(g * L, L)] = acc

    @pl.when(cid == 0)
    def _():
        pltpu.sync_copy(red, ph_hbm.at[sid])

    @pl.when(cid == 1)
    def _():
        pltpu.sync_copy(red, tp_hbm.at[sid])


_mesh = plsc.VectorSubcoreMesh(core_axis_name="c", subcore_axis_name="s",
                               num_cores=NC, num_subcores=NS)

_sc_hist = pl.kernel(
    _sc_body,
    out_type=(jax.ShapeDtypeStruct((ROWS, PAD), jnp.float32),
              jax.ShapeDtypeStruct((ROWS, PAD), jnp.float32)),
    mesh=_mesh,
    compiler_params=pltpu.CompilerParams(needs_layout_passes=False),
    scratch_types=[
        pltpu.VMEM((CH,), jnp.float32),
        pltpu.VMEM((CH,), jnp.float32),
        pltpu.VMEM((PAD * L,), jnp.float32),
        pltpu.VMEM((PAD,), jnp.float32),
        pltpu.VMEM((L,), jnp.float32),
        pltpu.VMEM_SHARED((2, NS, CH), jnp.float32),
        pltpu.SemaphoreType.DMA,
        pltpu.SemaphoreType.DMA,
        pltpu.SemaphoreType.DMA,
        pltpu.SemaphoreType.DMA,
    ],
)


def _loss_body(ph_ref, tp_ref, o_ref):
    ph = ph_ref[...]
    tp = tp_ref[...]
    cols = lax.broadcasted_iota(jnp.int32, (ROWS, PAD), 1)

    def fold(h):
        over = jnp.sum(jnp.where(cols == NBINS, h, 0.0), axis=1, keepdims=True)
        h = jnp.where(cols == NBINS - 1, h + over, h)
        return jnp.where(cols < NBINS, h, 0.0)

    ph = fold(ph)
    tp = fold(tp)
    th = jnp.sum(tp, axis=0, keepdims=True)
    base = jnp.minimum(ph, th)
    safe = jnp.where(ph == 0.0, 1.0, ph)
    r = base / safe
    sim = jnp.sum(r * r, axis=1) / jnp.float32(NBINS)
    o_ref[0] = jnp.sum(1.0 - sim)


_loss_tc = pl.pallas_call(
    _loss_body,
    out_shape=jax.ShapeDtypeStruct((1,), jnp.float32),
    out_specs=pl.BlockSpec(memory_space=pltpu.SMEM),
)


def kernel(output, target):
    lo, scale = _mm_tc(output, target)
    ph, tp = _sc_hist(output, target, lo, scale)
    loss = _loss_tc(ph, tp)
    return jnp.reshape(loss, ())


# speedup vs baseline: 1.1176x; 1.1176x over previous
import jax
import jax.numpy as jnp
from jax import lax
from jax.experimental import pallas as pl
from jax.experimental.pallas import tpu as pltpu
from jax.experimental.pallas import tpu_sc as plsc

NC = 2
NS = 16
L = 16
ROWS = 16
COLS = 1048576
HALF = COLS // 2
NBINS = 100
PAD = 128
CH = 32768
NCH = HALF // CH


MMW = 65536
MMG = COLS // MMW


def _make_mm(glob):
    def body(x_ref, lo_ref, scale_ref, amin, amax):
        i = pl.program_id(0)

        @pl.when(i == 0)
        def _():
            amin[...] = jnp.full((ROWS, 128), jnp.inf, jnp.float32)
            amax[...] = jnp.full((ROWS, 128), -jnp.inf, jnp.float32)

        x = x_ref[...]
        bc = lambda v: jnp.broadcast_to(v, (ROWS, 128))
        amin[...] = jnp.minimum(amin[...], bc(x.min(axis=1, keepdims=True)))
        amax[...] = jnp.maximum(amax[...], bc(x.max(axis=1, keepdims=True)))

        @pl.when(i == MMG - 1)
        def _():
            if glob:
                ones = jnp.ones((ROWS, 128), jnp.float32)
                lo = amin[...].min() * ones
                hi = amax[...].max() * ones
            else:
                lo = amin[...]
                hi = amax[...]
            lo_ref[...] = lo
            scale_ref[...] = jnp.float32(NBINS) / (hi - lo)

    return pl.pallas_call(
        body,
        grid=(MMG,),
        in_specs=[pl.BlockSpec((ROWS, MMW), lambda i: (0, i))],
        out_specs=[
            pl.BlockSpec((ROWS, 128), lambda i: (0, 0)),
            pl.BlockSpec((ROWS, 128), lambda i: (0, 0)),
        ],
        out_shape=[
            jax.ShapeDtypeStruct((ROWS, 128), jnp.float32),
            jax.ShapeDtypeStruct((ROWS, 128), jnp.float32),
        ],
        scratch_shapes=[pltpu.VMEM((ROWS, 128), jnp.float32)] * 2,
    )


_mm_row = _make_mm(glob=False)
_mm_glob = _make_mm(glob=True)


def _sc_body(x_hbm, lo_hbm, scale_hbm, out_hbm,
             buf0, buf1, buf2, hist, red, prm, sem0, sem1, sem2):
    cid = lax.axis_index("c")
    sid = lax.axis_index("s")
    bufs = (buf0, buf1, buf2)
    sems = (sem0, sem1, sem2)
    wid = cid * NS + sid
    row = wid >> 1
    half = wid & 1
    base = half * HALF

    pltpu.sync_copy(lo_hbm.at[row, pl.ds(0, L)], prm)
    lo = prm[...]
    pltpu.sync_copy(scale_hbm.at[row, pl.ds(0, L)], prm)
    scale = prm[...]

    zvec = jnp.zeros((L,), jnp.float32)

    @plsc.parallel_loop(0, PAD, unroll=8)
    def _(k):
        hist[pl.ds(pl.multiple_of(k * L, 8), L)] = zvec

    lane = lax.broadcasted_iota(jnp.int32, (L,), 0)
    ones = jnp.ones((L,), jnp.float32)

    def process(buf):
        @plsc.parallel_loop(0, CH // L, unroll=8)
        def _(i):
            x = buf[pl.ds(pl.multiple_of(i * L, 8), L)]
            t = (x - lo) * scale
            b = t.astype(jnp.int32)
            addr = (b << 4) | lane
            plsc.addupdate_scatter(hist, [addr], ones)

    def start(j, b):
        off = pl.multiple_of(base + j * CH, 8)
        return pltpu.async_copy(x_hbm.at[row, pl.ds(off, CH)],
                                bufs[b], sems[b])

    nbuf = len(bufs)
    descs = [start(j, j) for j in range(nbuf)]
    for j in range(NCH):
        b = j % nbuf
        descs[b].wait()
        process(bufs[b])
        if j + nbuf < NCH:
            descs[b] = start(j + nbuf, b)

    kidx = lax.broadcasted_iota(jnp.int32, (L,), 0) * L
    for g in range(PAD // L):
        acc = jnp.zeros((L,), jnp.float32)
        for l in range(L):
            acc = acc + plsc.load_gather(hist, [kidx + (g * L * L + l)])
        red[pl.ds(g * L, L)] = acc

    pltpu.sync_copy(red, out_hbm.at[half, row])


_mesh = plsc.VectorSubcoreMesh(core_axis_name="c", subcore_axis_name="s",
                               num_cores=NC, num_subcores=NS)

_sc_hist = pl.kernel(
    _sc_body,
    out_type=jax.ShapeDtypeStruct((2, ROWS, PAD), jnp.float32),
    mesh=_mesh,
    compiler_params=pltpu.CompilerParams(needs_layout_passes=False),
    scratch_types=[
        pltpu.VMEM((CH,), jnp.float32),
        pltpu.VMEM((CH,), jnp.float32),
        pltpu.VMEM((CH,), jnp.float32),
        pltpu.VMEM((PAD * L,), jnp.float32),
        pltpu.VMEM((PAD,), jnp.float32),
        pltpu.VMEM((L,), jnp.float32),
        pltpu.SemaphoreType.DMA,
        pltpu.SemaphoreType.DMA,
        pltpu.SemaphoreType.DMA,
    ],
)


def _loss_body(ph_ref, tp_ref, o_ref):
    ph = ph_ref[0] + ph_ref[1]
    tp = tp_ref[0] + tp_ref[1]
    cols = lax.broadcasted_iota(jnp.int32, (ROWS, PAD), 1)

    def fold(h):
        over = jnp.sum(jnp.where(cols == NBINS, h, 0.0), axis=1, keepdims=True)
        h = jnp.where(cols == NBINS - 1, h + over, h)
        return jnp.where(cols < NBINS, h, 0.0)

    ph = fold(ph)
    tp = fold(tp)
    th = jnp.sum(tp, axis=0, keepdims=True)
    base = jnp.minimum(ph, th)
    safe = jnp.where(ph == 0.0, 1.0, ph)
    r = base / safe
    sim = jnp.sum(r * r, axis=1) / jnp.float32(NBINS)
    o_ref[0] = jnp.sum(1.0 - sim)


_loss_tc = pl.pallas_call(
    _loss_body,
    out_shape=jax.ShapeDtypeStruct((1,), jnp.float32),
    out_specs=pl.BlockSpec(memory_space=pltpu.SMEM),
)


def kernel(output, target):
    tlo, tscale = _mm_glob(target)
    tp = _sc_hist(target, tlo, tscale)
    olo, oscale = _mm_row(output)
    ph = _sc_hist(output, olo, oscale)
    loss = _loss_tc(ph, tp)
    return jnp.reshape(loss, ())
